# Initial kernel scaffold; baseline (speedup 1.0000x reference)
import jax
import jax.numpy as jnp
from jax.experimental import pallas as pl


def kernel(h, edge_index, e, W1, b1, W2, b2, gamma, beta):
    raise NotImplementedError("write your pallas kernel here")


# speedup vs baseline: 3.7177x; 3.7177x over previous
import functools

import jax
import jax.numpy as jnp
from jax import lax
from jax.experimental import pallas as pl
from jax.experimental.pallas import tpu as pltpu
from jax.experimental.pallas import tpu_sc as plsc

N_NODES = 10000
N_EDGES = 320000
D = 128
NC, NS = 2, 16
NW = NC * NS
EPW = N_EDGES // NW
CHUNK = 80
NCHUNKS = EPW // CHUNK
NPAD = 10240
ROWS_PER_TILE = NPAD // NS
ZROWS = 128
EPS = 1e-5

_mesh = plsc.VectorSubcoreMesh(
    core_axis_name="c", subcore_axis_name="s", num_cores=NC, num_subcores=NS
)


CROWS = NPAD // D


@functools.partial(
    pl.kernel,
    out_type=(
        jax.ShapeDtypeStruct((NC, NPAD, D), jnp.float32),
        jax.ShapeDtypeStruct((NC, CROWS, D), jnp.float32),
    ),
    mesh=_mesh,
    scratch_types=[
        pltpu.VMEM((CHUNK,), jnp.int32),
        pltpu.VMEM((CHUNK,), jnp.int32),
        pltpu.VMEM((CHUNK,), jnp.int32),
        pltpu.VMEM((CHUNK,), jnp.int32),
        pltpu.VMEM((CHUNK,), jnp.float32),
        pltpu.VMEM((CHUNK, D), jnp.float32),
        pltpu.VMEM((CHUNK, D), jnp.float32),
        pltpu.VMEM((ZROWS, D), jnp.float32),
        pltpu.VMEM_SHARED((D, D), jnp.float32),
        pltpu.VMEM_SHARED((NPAD, D), jnp.float32),
        pltpu.VMEM_SHARED((CROWS, D), jnp.float32),
        pltpu.SemaphoreType.DMA,
        pltpu.SemaphoreType.DMA,
    ],
)
def _sc_aggregate(h_hbm, src_hbm, dst_hbm, e_hbm, outf_hbm, outc_hbm,
                  src_v, dst_v, drow_v, dcol_v, e_v, rows_v, oh_v,
                  zero_v, tab_sh, acc_sh, cnt_sh, sem, sem2):
    cid = lax.axis_index("c")
    sid = lax.axis_index("s")
    wid = sid * NC + cid

    zero16 = jnp.zeros((16,), jnp.float32)
    lanes = lax.iota(jnp.int32, 16)

    @pl.loop(0, ZROWS)
    def _zinit(r):
        for j in range(D // 16):
            zero_v[r, pl.ds(16 * j, 16)] = zero16

    @pl.loop(0, D)
    def _tinit(r):
        rv = jnp.full((16,), r)
        for j in range(D // 16):
            oh_v[0, pl.ds(16 * j, 16)] = jnp.where(
                lanes + (16 * j) == rv, jnp.float32(1.0), jnp.float32(0.0)
            )
        @pl.when(sid == 0)
        def _():
            pltpu.sync_copy(oh_v.at[pl.ds(0, 1)], tab_sh.at[pl.ds(r, 1)])

    for i in range(ROWS_PER_TILE // ZROWS):
        pltpu.sync_copy(
            zero_v, acc_sh.at[pl.ds(sid * ROWS_PER_TILE + i * ZROWS, ZROWS)]
        )

    @pl.when(sid == 0)
    def _():
        pltpu.sync_copy(zero_v.at[pl.ds(0, CROWS)], cnt_sh)

    plsc.subcore_barrier()

    ebase = wid * EPW

    @pl.loop(0, NCHUNKS)
    def _chunk(c):
        base = pl.multiple_of(ebase + c * CHUNK, 8)
        pltpu.sync_copy(src_hbm.at[pl.ds(base, CHUNK)], src_v)
        pltpu.sync_copy(dst_hbm.at[pl.ds(base, CHUNK)], dst_v)
        pltpu.sync_copy(e_hbm.at[pl.ds(base, CHUNK)], e_v)
        pltpu.async_copy(h_hbm.at[src_v], rows_v, sem).wait()

        @pl.loop(0, CHUNK // 16)
        def _grp(g):
            evec = e_v[pl.ds(g * 16, 16)]
            dvec = dst_v[pl.ds(g * 16, 16)]
            drow_v[pl.ds(g * 16, 16)] = lax.shift_right_logical(dvec, 7)
            dcol_v[pl.ds(g * 16, 16)] = lax.bitwise_and(dvec, 127)
            kbase = g * 16
            for l in range(16):
                ev = jnp.full((16,), evec[l])
                for j in range(D // 16):
                    rows_v[kbase + l, pl.ds(16 * j, 16)] = (
                        rows_v[kbase + l, pl.ds(16 * j, 16)] * ev
                    )

        pltpu.async_copy(tab_sh.at[dcol_v], oh_v, sem2).wait()
        pltpu.sync_copy(rows_v, acc_sh.at[dst_v], add=True)
        pltpu.sync_copy(oh_v, cnt_sh.at[drow_v], add=True)

    plsc.subcore_barrier()

    pltpu.sync_copy(
        acc_sh.at[pl.ds(sid * ROWS_PER_TILE, ROWS_PER_TILE)],
        outf_hbm.at[cid, pl.ds(sid * ROWS_PER_TILE, ROWS_PER_TILE)],
    )

    @pl.when(sid == 0)
    def _():
        pltpu.sync_copy(cnt_sh, outc_hbm.at[cid])


_TCROWS = 1000
_TCGRID = N_NODES // _TCROWS


def _mlp_body(acc_ref, cnt_ref, w1_ref, b1_ref, w2_ref, b2_ref,
              agg_ref, x_ref, stats_ref):
    i = pl.program_id(0)
    summ = acc_ref[0] + acc_ref[1]
    cnt = cnt_ref[...]
    agg = summ / jnp.maximum(cnt, 1.0)
    x = jnp.dot(agg, w1_ref[...], preferred_element_type=jnp.float32,
                precision=lax.Precision.HIGHEST) + b1_ref[...]
    x = jnp.maximum(x, 0.0)
    x = jnp.dot(x, w2_ref[...], preferred_element_type=jnp.float32,
                precision=lax.Precision.HIGHEST) + b2_ref[...]
    x = jnp.maximum(x, 0.0)
    agg_ref[...] = agg
    x_ref[...] = x

    @pl.when(i == 0)
    def _():
        stats_ref[...] = jnp.zeros_like(stats_ref)

    stats_ref[0:1, :] += jnp.sum(x, axis=0, keepdims=True)
    stats_ref[1:2, :] += jnp.sum(x * x, axis=0, keepdims=True)


def _bn_body(x_ref, agg_ref, stats_ref, gamma_ref, beta_ref, out_ref):
    mu = stats_ref[0:1, :] / N_NODES
    var = stats_ref[1:2, :] / N_NODES - mu * mu
    inv = lax.rsqrt(var + EPS)
    out_ref[...] = ((x_ref[...] - mu) * inv * gamma_ref[...]
                    + beta_ref[...] + agg_ref[...])


def kernel(h, edge_index, e, W1, b1, W2, b2, gamma, beta):
    src = edge_index[0]
    dst = edge_index[1]
    ew = e[:, 0]

    acc, acc_cnt = _sc_aggregate(h, src, dst, ew)
    cnt = (acc_cnt[0] + acc_cnt[1]).reshape(NPAD, 1)

    w1t = W1.T
    w2t = W2.T
    b1r = b1.reshape(1, D)
    b2r = b2.reshape(1, D)
    gr = gamma.reshape(1, D)
    br = beta.reshape(1, D)

    agg, x, stats = pl.pallas_call(
        _mlp_body,
        grid=(_TCGRID,),
        in_specs=[
            pl.BlockSpec((NC, _TCROWS, D), lambda i: (0, i, 0)),
            pl.BlockSpec((_TCROWS, 1), lambda i: (i, 0)),
            pl.BlockSpec((D, D), lambda i: (0, 0)),
            pl.BlockSpec((1, D), lambda i: (0, 0)),
            pl.BlockSpec((D, D), lambda i: (0, 0)),
            pl.BlockSpec((1, D), lambda i: (0, 0)),
        ],
        out_specs=[
            pl.BlockSpec((_TCROWS, D), lambda i: (i, 0)),
            pl.BlockSpec((_TCROWS, D), lambda i: (i, 0)),
            pl.BlockSpec((2, D), lambda i: (0, 0)),
        ],
        out_shape=[
            jax.ShapeDtypeStruct((N_NODES, D), jnp.float32),
            jax.ShapeDtypeStruct((N_NODES, D), jnp.float32),
            jax.ShapeDtypeStruct((2, D), jnp.float32),
        ],
    )(acc, cnt, w1t, b1r, w2t, b2r)

    out = pl.pallas_call(
        _bn_body,
        grid=(_TCGRID,),
        in_specs=[
            pl.BlockSpec((_TCROWS, D), lambda i: (i, 0)),
            pl.BlockSpec((_TCROWS, D), lambda i: (i, 0)),
            pl.BlockSpec((2, D), lambda i: (0, 0)),
            pl.BlockSpec((1, D), lambda i: (0, 0)),
            pl.BlockSpec((1, D), lambda i: (0, 0)),
        ],
        out_specs=pl.BlockSpec((_TCROWS, D), lambda i: (i, 0)),
        out_shape=jax.ShapeDtypeStruct((N_NODES, D), jnp.float32),
    )(x, agg, stats, gr, br)

    return out
